# baseline (speedup 1.0000x reference)
import functools

import jax
import jax.numpy as jnp
from jax import lax
from jax.experimental import pallas as pl
from jax.experimental.pallas import tpu as pltpu
from jax.experimental.pallas import tpu_sc as plsc

D_MODEL = 64
SCALE = float(D_MODEL) ** 0.5
NC, NS = 2, 16
NW = NC * NS
NBUF = 4


def _scale_chunk(rows, n_rows):
    def body(r, carry):
        for c in range(D_MODEL // 16):
            sl = (r, pl.ds(c * 16, 16))
            rows[sl] = rows[sl] * SCALE
        return carry
    lax.fori_loop(0, n_rows, body, 0, unroll=5)


@functools.cache
def _make_kernel(n_seq, seq_len):
    steps = n_seq // NW
    assert steps % NBUF == 0 and steps >= 2 * NBUF
    pad_rows = (seq_len + 7) // 8 * 8
    pad_cols = 128

    mesh = plsc.VectorSubcoreMesh(core_axis_name="c", subcore_axis_name="s")

    @functools.partial(
        pl.kernel,
        mesh=mesh,
        out_type=jax.ShapeDtypeStruct((n_seq, pad_rows, pad_cols), jnp.float32),
        scratch_types=(
            [pltpu.VMEM((steps, seq_len), jnp.int32)]
            + [pltpu.VMEM((seq_len, pad_cols), jnp.float32)] * NBUF
            + [pltpu.SemaphoreType.DMA] * (2 * NBUF)
        ),
        compiler_params=pltpu.CompilerParams(use_tc_tiling_on_sc=False),
    )
    def emb(idx_hbm, table_hbm, out_hbm, idx_v, r0, r1, r2, r3,
            g0, g1, g2, g3, s0, s1, s2, s3):
        bufs = (r0, r1, r2, r3)
        gsems = (g0, g1, g2, g3)
        ssems = (s0, s1, s2, s3)
        wid = lax.axis_index("s") * NC + lax.axis_index("c")
        row0 = wid * steps

        pltpu.sync_copy(idx_hbm.at[pl.ds(row0, steps)], idx_v)

        def g_start(s, b):
            pltpu.make_async_copy(
                table_hbm.at[idx_v.at[s]], bufs[b], gsems[b]).start()

        def g_wait(s, b):
            pltpu.make_async_copy(
                table_hbm.at[idx_v.at[s]], bufs[b], gsems[b]).wait()

        def st_start(s, b):
            pltpu.make_async_copy(
                bufs[b],
                out_hbm.at[row0 + s, pl.ds(0, seq_len)],
                ssems[b]).start()

        def st_wait(s, b):
            pltpu.make_async_copy(
                bufs[b],
                out_hbm.at[row0 + s, pl.ds(0, seq_len)],
                ssems[b]).wait()

        g_start(0, 0)
        g_start(1, 1)

        g_start(2, 2)
        g_wait(0, 0)
        _scale_chunk(bufs[0], seq_len)
        st_start(0, 0)

        g_start(3, 3)
        g_wait(1, 1)
        _scale_chunk(bufs[1], seq_len)
        st_start(1, 1)

        def body(i, carry):
            for k in range(NBUF):
                s = 2 + i * NBUF + k
                b = (2 + k) % NBUF
                b2 = k % NBUF
                st_wait(s - 2, b2)
                g_start(s + 2, b2)
                g_wait(s, b)
                _scale_chunk(bufs[b], seq_len)
                st_start(s, b)
            return carry
        lax.fori_loop(0, (steps - 4) // NBUF, body, 0)

        g_wait(steps - 2, 2)
        _scale_chunk(bufs[2], seq_len)
        st_start(steps - 2, 2)

        g_wait(steps - 1, 3)
        _scale_chunk(bufs[3], seq_len)
        st_start(steps - 1, 3)

        st_wait(steps - 4, 0)
        st_wait(steps - 3, 1)
        st_wait(steps - 2, 2)
        st_wait(steps - 1, 3)

    return emb


def kernel(x, table):
    n_seq, seq_len = x.shape
    tp = jnp.pad(table, ((0, 0), (0, 128 - D_MODEL)))
    padded = _make_kernel(n_seq, seq_len)(x.astype(jnp.int32), tp)
    return padded[:, :seq_len, :D_MODEL]


# speedup vs baseline: 1.0473x; 1.0473x over previous
import functools

import jax
import jax.numpy as jnp
from jax import lax
from jax.experimental import pallas as pl
from jax.experimental.pallas import tpu as pltpu
from jax.experimental.pallas import tpu_sc as plsc

D_MODEL = 64
SCALE = float(D_MODEL) ** 0.5
NC, NS = 2, 16
NW = NC * NS
NBUF = 4


def _scale_chunk(rows, n_rows):
    def body(r, carry):
        for c in range(D_MODEL // 16):
            sl = (r, pl.ds(c * 16, 16))
            rows[sl] = rows[sl] * SCALE
        return carry
    lax.fori_loop(0, n_rows, body, 0, unroll=5)


@functools.cache
def _make_kernel(n_seq, seq_len):
    steps = n_seq // NW
    assert steps % NBUF == 0 and steps >= 2 * NBUF
    pad_rows = (seq_len + 7) // 8 * 8
    pad_cols = 128

    mesh = plsc.VectorSubcoreMesh(core_axis_name="c", subcore_axis_name="s")

    @functools.partial(
        pl.kernel,
        mesh=mesh,
        out_type=jax.ShapeDtypeStruct((n_seq, pad_rows, pad_cols), jnp.float32),
        scratch_types=(
            [pltpu.VMEM((steps, seq_len), jnp.int32)]
            + [pltpu.VMEM((seq_len, D_MODEL), jnp.float32)] * NBUF
            + [pltpu.SemaphoreType.DMA] * (2 * NBUF)
        ),
        compiler_params=pltpu.CompilerParams(use_tc_tiling_on_sc=False),
    )
    def emb(idx_hbm, table_hbm, out_hbm, idx_v, r0, r1, r2, r3,
            g0, g1, g2, g3, s0, s1, s2, s3):
        bufs = (r0, r1, r2, r3)
        gsems = (g0, g1, g2, g3)
        ssems = (s0, s1, s2, s3)
        wid = lax.axis_index("s") * NC + lax.axis_index("c")
        row0 = wid * steps

        pltpu.sync_copy(idx_hbm.at[pl.ds(row0, steps)], idx_v)

        def g_start(s, b):
            pltpu.make_async_copy(
                table_hbm.at[idx_v.at[s]], bufs[b], gsems[b]).start()

        def g_wait(s, b):
            pltpu.make_async_copy(
                table_hbm.at[idx_v.at[s]], bufs[b], gsems[b]).wait()

        def st_start(s, b):
            pltpu.make_async_copy(
                bufs[b],
                out_hbm.at[row0 + s, pl.ds(0, seq_len), pl.ds(0, D_MODEL)],
                ssems[b]).start()

        def st_wait(s, b):
            pltpu.make_async_copy(
                bufs[b],
                out_hbm.at[row0 + s, pl.ds(0, seq_len), pl.ds(0, D_MODEL)],
                ssems[b]).wait()

        g_start(0, 0)
        g_start(1, 1)

        g_start(2, 2)
        g_wait(0, 0)
        _scale_chunk(bufs[0], seq_len)
        st_start(0, 0)

        g_start(3, 3)
        g_wait(1, 1)
        _scale_chunk(bufs[1], seq_len)
        st_start(1, 1)

        def body(i, carry):
            for k in range(NBUF):
                s = 2 + i * NBUF + k
                b = (2 + k) % NBUF
                b2 = k % NBUF
                st_wait(s - 2, b2)
                g_start(s + 2, b2)
                g_wait(s, b)
                _scale_chunk(bufs[b], seq_len)
                st_start(s, b)
            return carry
        lax.fori_loop(0, (steps - 4) // NBUF, body, 0)

        g_wait(steps - 2, 2)
        _scale_chunk(bufs[2], seq_len)
        st_start(steps - 2, 2)

        g_wait(steps - 1, 3)
        _scale_chunk(bufs[3], seq_len)
        st_start(steps - 1, 3)

        st_wait(steps - 4, 0)
        st_wait(steps - 3, 1)
        st_wait(steps - 2, 2)
        st_wait(steps - 1, 3)

    return emb


def kernel(x, table):
    n_seq, seq_len = x.shape
    padded = _make_kernel(n_seq, seq_len)(x.astype(jnp.int32), table)
    return padded[:, :seq_len, :D_MODEL]


# speedup vs baseline: 1.0942x; 1.0448x over previous
import functools

import jax
import jax.numpy as jnp
from jax import lax
from jax.experimental import pallas as pl
from jax.experimental.pallas import tpu as pltpu
from jax.experimental.pallas import tpu_sc as plsc

D_MODEL = 64
SCALE = float(D_MODEL) ** 0.5
NC, NS = 2, 16
NW = NC * NS
NBUF = 8
AHEAD = 4


def _scale_chunk(rows, n_rows):
    def body(r, carry):
        for c in range(D_MODEL // 16):
            sl = (r, pl.ds(c * 16, 16))
            rows[sl] = rows[sl] * SCALE
        return carry
    lax.fori_loop(0, n_rows, body, 0, unroll=5)


@functools.cache
def _make_kernel(n_seq, seq_len):
    steps = n_seq // NW
    assert steps % NBUF == 0 and steps >= 2 * NBUF
    pad_rows = (seq_len + 7) // 8 * 8
    pad_cols = 128

    mesh = plsc.VectorSubcoreMesh(core_axis_name="c", subcore_axis_name="s")

    @functools.partial(
        pl.kernel,
        mesh=mesh,
        out_type=jax.ShapeDtypeStruct((n_seq, pad_rows, pad_cols), jnp.float32),
        scratch_types=(
            [pltpu.VMEM((steps, seq_len), jnp.int32)]
            + [pltpu.VMEM((seq_len, D_MODEL), jnp.float32)] * NBUF
            + [pltpu.SemaphoreType.DMA] * (2 * NBUF)
        ),
        compiler_params=pltpu.CompilerParams(use_tc_tiling_on_sc=False),
    )
    def emb(idx_hbm, table_hbm, out_hbm, idx_v, *rest):
        bufs = rest[:NBUF]
        gsems = rest[NBUF:2 * NBUF]
        ssems = rest[2 * NBUF:]
        wid = lax.axis_index("s") * NC + lax.axis_index("c")
        row0 = wid * steps

        pltpu.sync_copy(idx_hbm.at[pl.ds(row0, steps)], idx_v)

        def g_start(s, b):
            pltpu.make_async_copy(
                table_hbm.at[idx_v.at[s]], bufs[b], gsems[b]).start()

        def g_wait(s, b):
            pltpu.make_async_copy(
                table_hbm.at[idx_v.at[s]], bufs[b], gsems[b]).wait()

        def st_start(s, b):
            pltpu.make_async_copy(
                bufs[b],
                out_hbm.at[row0 + s, pl.ds(0, seq_len), pl.ds(0, D_MODEL)],
                ssems[b]).start()

        def st_wait(s, b):
            pltpu.make_async_copy(
                bufs[b],
                out_hbm.at[row0 + s, pl.ds(0, seq_len), pl.ds(0, D_MODEL)],
                ssems[b]).wait()

        for s in range(AHEAD):
            g_start(s, s)

        for s in range(AHEAD):
            g_start(s + AHEAD, (s + AHEAD) % NBUF)
            g_wait(s, s)
            _scale_chunk(bufs[s], seq_len)
            st_start(s, s)

        def body(i, carry):
            for k in range(NBUF):
                s = AHEAD + i * NBUF + k
                b = (AHEAD + k) % NBUF
                b2 = k % NBUF
                st_wait(s + AHEAD - NBUF, b2)
                g_start(s + AHEAD, b2)
                g_wait(s, b)
                _scale_chunk(bufs[b], seq_len)
                st_start(s, b)
            return carry
        lax.fori_loop(0, (steps - 2 * AHEAD) // NBUF, body, 0)

        for j in range(AHEAD):
            s = steps - AHEAD + j
            b = s % NBUF
            g_wait(s, b)
            _scale_chunk(bufs[b], seq_len)
            st_start(s, b)

        for j in range(NBUF):
            s = steps - NBUF + j
            st_wait(s, s % NBUF)

    return emb


def kernel(x, table):
    n_seq, seq_len = x.shape
    padded = _make_kernel(n_seq, seq_len)(x.astype(jnp.int32), table)
    return padded[:, :seq_len, :D_MODEL]
